# baseline (speedup 1.0000x reference)
import functools

import jax
import jax.numpy as jnp
from jax import lax
from jax.experimental import pallas as pl
from jax.experimental.pallas import tpu as pltpu


def _round_up(v, m):
    return (v + m - 1) // m * m


def _pad2d(a, rows, cols):
    r, c = a.shape
    if r == rows and c == cols:
        return a
    return jnp.pad(a, ((0, rows - r), (0, cols - c)))


def _sample_kernel(mu_ref, rho_ref, eps_ref, w_ref):
    sigma = jax.nn.softplus(rho_ref[...]) + 1e-5
    w_ref[...] = (mu_ref[...] + eps_ref[...] * sigma).astype(jnp.bfloat16)


def _matmul_kernel(x_ref, w_ref, b_ref, o_ref):
    xb = x_ref[...].astype(jnp.bfloat16)
    acc = lax.dot_general(
        xb, w_ref[...],
        dimension_numbers=(((1,), (1,)), ((), ())),
        preferred_element_type=jnp.float32)
    o_ref[...] = acc + b_ref[...]


def _sample_weights(mu, rho, eps, Np, Kp, tn):
    return pl.pallas_call(
        _sample_kernel,
        out_shape=jax.ShapeDtypeStruct((Np, Kp), jnp.bfloat16),
        grid=(Np // tn,),
        in_specs=[pl.BlockSpec((tn, Kp), lambda j: (j, 0))] * 3,
        out_specs=pl.BlockSpec((tn, Kp), lambda j: (j, 0)),
        compiler_params=pltpu.CompilerParams(
            dimension_semantics=("parallel",),
            vmem_limit_bytes=100 * 2**20),
    )(mu, rho, eps)


def _forward(x, w, bias2d, Bp, Np, Kp, tm):
    return pl.pallas_call(
        _matmul_kernel,
        out_shape=jax.ShapeDtypeStruct((Bp, Np), jnp.float32),
        grid=(Bp // tm,),
        in_specs=[
            pl.BlockSpec((tm, Kp), lambda i: (i, 0)),
            pl.BlockSpec((Np, Kp), lambda i: (0, 0)),
            pl.BlockSpec((1, Np), lambda i: (0, 0)),
        ],
        out_specs=pl.BlockSpec((tm, Np), lambda i: (i, 0)),
        compiler_params=pltpu.CompilerParams(
            dimension_semantics=("parallel",),
            vmem_limit_bytes=100 * 2**20),
    )(x, w, bias2d)


@jax.jit
def kernel(x, mu, rho, eps, bias):
    B, in_f = x.shape
    out_f, _ = mu.shape

    x = x.astype(jnp.float32)
    mu = mu.astype(jnp.float32)
    rho = rho.astype(jnp.float32)
    eps = eps.astype(jnp.float32)
    bias = bias.astype(jnp.float32)

    Bp = _round_up(B, 256)
    Np = _round_up(out_f, 256)
    Kp = _round_up(in_f, 256)

    xp = _pad2d(x, Bp, Kp)
    mup = _pad2d(mu, Np, Kp)
    rhop = _pad2d(rho, Np, Kp)
    epsp = _pad2d(eps, Np, Kp)
    biasp = _pad2d(bias.reshape(1, out_f), 1, Np)

    tn = 128 if Np % 1024 == 0 else Np
    w = _sample_weights(mup, rhop, epsp, Np, Kp, tn)

    tm = 512 if Bp % 512 == 0 else Bp
    out = _forward(xp, w, biasp, Bp, Np, Kp, tm)

    if Bp != B or Np != out_f:
        out = out[:B, :out_f]
    return out


# speedup vs baseline: 1.3839x; 1.3839x over previous
import jax
import jax.numpy as jnp
from jax import lax
from jax.experimental import pallas as pl
from jax.experimental.pallas import tpu as pltpu

_LOG2E = 1.4426950408889634
_LN2 = 0.6931471805599453


def _round_up(v, m):
    return (v + m - 1) // m * m


def _pad2d(a, rows, cols):
    r, c = a.shape
    if r == rows and c == cols:
        return a
    return jnp.pad(a, ((0, rows - r), (0, cols - c)))


def _fused_kernel(x_ref, mu_ref, rho_ref, eps_ref, b_ref, o_ref):
    t = jnp.exp2(rho_ref[...] * _LOG2E)
    sigma = jnp.log2(1.0 + t) * _LN2 + 1e-5
    w = (mu_ref[...] + eps_ref[...] * sigma).astype(jnp.bfloat16)
    xb = x_ref[...].astype(jnp.bfloat16)
    acc = lax.dot_general(
        xb, w,
        dimension_numbers=(((1,), (1,)), ((), ())),
        preferred_element_type=jnp.float32)
    o_ref[...] = acc + b_ref[...]


def _forward(x, mu, rho, eps, bias2d, Bp, Np, Kp, tm):
    return pl.pallas_call(
        _fused_kernel,
        out_shape=jax.ShapeDtypeStruct((Bp, Np), jnp.float32),
        grid=(Bp // tm,),
        in_specs=[
            pl.BlockSpec((tm, Kp), lambda i: (i, 0)),
            pl.BlockSpec((Np, Kp), lambda i: (0, 0)),
            pl.BlockSpec((Np, Kp), lambda i: (0, 0)),
            pl.BlockSpec((Np, Kp), lambda i: (0, 0)),
            pl.BlockSpec((1, Np), lambda i: (0, 0)),
        ],
        out_specs=pl.BlockSpec((tm, Np), lambda i: (i, 0)),
        compiler_params=pltpu.CompilerParams(
            dimension_semantics=("parallel",),
            vmem_limit_bytes=100 * 2**20),
    )(x, mu, rho, eps, bias2d)


@jax.jit
def kernel(x, mu, rho, eps, bias):
    B, in_f = x.shape
    out_f, _ = mu.shape

    x = x.astype(jnp.float32)
    mu = mu.astype(jnp.float32)
    rho = rho.astype(jnp.float32)
    eps = eps.astype(jnp.float32)
    bias = bias.astype(jnp.float32)

    Bp = _round_up(B, 256)
    Np = _round_up(out_f, 256)
    Kp = _round_up(in_f, 256)

    xp = _pad2d(x, Bp, Kp)
    mup = _pad2d(mu, Np, Kp)
    rhop = _pad2d(rho, Np, Kp)
    epsp = _pad2d(eps, Np, Kp)
    biasp = _pad2d(bias.reshape(1, out_f), 1, Np)

    tm = 1024 if Bp % 1024 == 0 else (512 if Bp % 512 == 0 else Bp)
    out = _forward(xp, mup, rhop, epsp, biasp, Bp, Np, Kp, tm)

    if Bp != B or Np != out_f:
        out = out[:B, :out_f]
    return out
